# baseline (speedup 1.0000x reference)
import dataclasses
import functools

import jax
import jax.numpy as jnp
from jax import lax
from jax.experimental import pallas as pl
from jax.experimental.pallas import tpu as pltpu
from jax.experimental.pallas import tpu_sc as plsc

_B, _T, _D = 4096, 50, 128
_N = _B * _T
_NWORKERS = 32
_TPW = _N // _NWORKERS
_W = 64
_NWIN = _TPW // _W

_DOSE_V, _TOTAL_V, _UNIT_V = 14, 512, 64

_DOSE_BOUNDS = (0.0, 0.1, 0.5, 1.0, 2.0, 5.0, 10.0, 20.0, 50.0, 100.0,
                200.0, 500.0, 1000.0)


def _composite_embedding(data_i, dose_f, total_i, unit_i,
                         data_table, dose_table, total_table, unit_table):
  mesh = plsc.VectorSubcoreMesh(core_axis_name="core",
                                subcore_axis_name="subcore")

  cparams = pltpu.CompilerParams()
  if "needs_layout_passes" in pltpu.CompilerParams.__dataclass_fields__:
    cparams = dataclasses.replace(cparams, needs_layout_passes=False)

  @functools.partial(
      pl.kernel,
      out_type=jax.ShapeDtypeStruct((_N, _D), jnp.float32),
      mesh=mesh,
      compiler_params=cparams,
      scratch_types=[
          pltpu.VMEM((_TPW,), jnp.int32),
          pltpu.VMEM((_TPW,), jnp.float32),
          pltpu.VMEM((_TPW,), jnp.int32),
          pltpu.VMEM((_TPW,), jnp.int32),
          pltpu.VMEM((_DOSE_V, _D), jnp.float32),
          pltpu.VMEM((_UNIT_V, _D), jnp.float32),
          pltpu.VMEM((_TOTAL_V, _D), jnp.float32),
          pltpu.VMEM((_W, _D), jnp.float32),
          pltpu.VMEM((_W, _D), jnp.float32),
          pltpu.SemaphoreType.DMA((2,)),
          pltpu.SemaphoreType.DMA((2,)),
      ],
  )
  def k(data_hbm, dose_hbm, total_hbm, unit_hbm,
        dtab_hbm, qtab_hbm, ttab_hbm, utab_hbm,
        out_hbm, di, dof, ti, ui, qtab, utab, ttab, ob0, ob1,
        sem_g, sem_o):
    wid = lax.axis_index("subcore") * 2 + lax.axis_index("core")
    base = wid * _TPW
    obs = (ob0, ob1)

    cp1 = pltpu.async_copy(data_hbm.at[pl.ds(base, _TPW)], di, sem_g.at[0])
    cp2 = pltpu.async_copy(dose_hbm.at[pl.ds(base, _TPW)], dof, sem_g.at[0])
    cp3 = pltpu.async_copy(total_hbm.at[pl.ds(base, _TPW)], ti, sem_g.at[0])
    cp4 = pltpu.async_copy(unit_hbm.at[pl.ds(base, _TPW)], ui, sem_g.at[0])
    cp5 = pltpu.async_copy(qtab_hbm, qtab, sem_g.at[1])
    cp6 = pltpu.async_copy(utab_hbm, utab, sem_g.at[1])
    cp7 = pltpu.async_copy(ttab_hbm, ttab, sem_g.at[1])
    for cp in (cp1, cp2, cp3, cp4, cp5, cp6, cp7):
      cp.wait()

    def fire(w, s):
      pltpu.make_async_copy(
          dtab_hbm.at[di.at[pl.ds(w * _W, _W)]], obs[s], sem_g.at[s]).start()

    def wait_gather(s):
      pltpu.make_async_copy(
          dtab_hbm.at[di.at[pl.ds(0, _W)]], obs[s], sem_g.at[s]).wait()

    def drain_out(s):
      pltpu.make_async_copy(
          obs[s], out_hbm.at[pl.ds(base, _W)], sem_o.at[s]).wait()

    fire(0, 0)

    @pl.loop(0, _NWIN // 2)
    def _(p):
      for s in (0, 1):
        w = 2 * p + s
        nxt = 1 - s

        @pl.when(w + 1 < _NWIN)
        def _():
          @pl.when(w >= 1)
          def _():
            drain_out(nxt)
          fire(w + 1, nxt)

        wait_gather(s)
        t0 = w * _W

        for g in range(_W // 16):
          off = t0 + g * 16
          d = dof[pl.ds(off, 16)]
          q_v = jnp.zeros((16,), jnp.int32)
          for bound in _DOSE_BOUNDS:
            q_v = q_v + jnp.where(d > bound, 1, 0).astype(jnp.int32)
          u_v = ui[pl.ds(off, 16)]
          t_v = ti[pl.ds(off, 16)]
          tok = lax.iota(jnp.int32, 16) + (g * 16)

          @pl.loop(0, _D, step=4)
          def _(j):
            for jj in range(4):
              col = jnp.full((16,), j + jj, jnp.int32)
              vo = plsc.load_gather(obs[s], [tok, col])
              vq = plsc.load_gather(qtab, [q_v, col])
              vu = plsc.load_gather(utab, [u_v, col])
              vt = plsc.load_gather(ttab, [t_v, col])
              plsc.store_scatter(obs[s], [tok, col], vo + vq + vu + vt)

        pltpu.make_async_copy(
            obs[s], out_hbm.at[pl.ds(base + t0, _W)], sem_o.at[s]).start()

    drain_out(0)
    drain_out(1)

  return k(data_i, dose_f, total_i, unit_i,
           data_table, dose_table, total_table, unit_table)


def kernel(data, dose, total, unit, data_table, dose_table, total_table,
           unit_table):
  out = _composite_embedding(
      data.reshape(_N), dose.reshape(_N), total.reshape(_N), unit.reshape(_N),
      data_table, dose_table, total_table, unit_table)
  return out.reshape(_B, _T, _D)


# speedup vs baseline: 1.5414x; 1.5414x over previous
import dataclasses
import functools

import jax
import jax.numpy as jnp
from jax import lax
from jax.experimental import pallas as pl
from jax.experimental.pallas import tpu as pltpu
from jax.experimental.pallas import tpu_sc as plsc

_B, _T, _D = 4096, 50, 128
_N = _B * _T
_NWORKERS = 32
_TPW = _N // _NWORKERS
_W = 128
_NWIN = _TPW // _W

_DOSE_BOUNDS = (0.0, 0.1, 0.5, 1.0, 2.0, 5.0, 10.0, 20.0, 50.0, 100.0,
                200.0, 500.0, 1000.0)


def _build_du_table(dose_table, unit_table):
  nq, nu = dose_table.shape[0], unit_table.shape[0]

  def body(q_ref, u_ref, o_ref):
    o_ref[...] = (q_ref[...][:, None, :]
                  + u_ref[...][None, :, :]).astype(jnp.bfloat16)

  out = pl.pallas_call(
      body,
      out_shape=jax.ShapeDtypeStruct((nq, nu, _D), jnp.bfloat16),
  )(dose_table, unit_table)
  return out.reshape(nq * nu, _D)


def _pack_bf16(table):
  v = table.shape[0]
  return jax.lax.bitcast_convert_type(
      table.reshape(v, _D // 2, 2), jnp.int32)


def _composite_embedding(data_i, dose_f, total_i, unit_i,
                         data_table, du_table, total_table):
  mesh = plsc.VectorSubcoreMesh(core_axis_name="core",
                                subcore_axis_name="subcore")

  cparams = pltpu.CompilerParams()
  if "needs_layout_passes" in pltpu.CompilerParams.__dataclass_fields__:
    cparams = dataclasses.replace(cparams, needs_layout_passes=False)
  if "use_tc_tiling_on_sc" in pltpu.CompilerParams.__dataclass_fields__:
    cparams = dataclasses.replace(cparams, use_tc_tiling_on_sc=False)

  @functools.partial(
      pl.kernel,
      out_type=jax.ShapeDtypeStruct((_N, _D // 2), jnp.int32),
      mesh=mesh,
      compiler_params=cparams,
      scratch_types=[
          pltpu.VMEM((_TPW,), jnp.int32),
          pltpu.VMEM((_TPW,), jnp.float32),
          pltpu.VMEM((_TPW,), jnp.int32),
          pltpu.VMEM((_TPW,), jnp.int32),
          pltpu.VMEM((2, _W), jnp.int32),
          pltpu.VMEM((2, _W, _D // 2), jnp.int32),
          pltpu.VMEM((2, _W, _D // 2), jnp.int32),
          pltpu.VMEM((2, _W, _D // 2), jnp.int32),
          pltpu.SemaphoreType.DMA((2,)),
          pltpu.SemaphoreType.DMA((2,)),
      ],
  )
  def k(data_hbm, dose_hbm, total_hbm, unit_hbm,
        dtab_hbm, dutab_hbm, ttab_hbm,
        out_hbm, di, dof, ti, ui, ci, bdu, bt, ob, sem_g, sem_o):
    wid = lax.axis_index("subcore") * 2 + lax.axis_index("core")
    base = wid * _TPW
    cp1 = pltpu.async_copy(data_hbm.at[pl.ds(base, _TPW)], di, sem_o.at[0])
    cp2 = pltpu.async_copy(dose_hbm.at[pl.ds(base, _TPW)], dof, sem_o.at[0])
    cp3 = pltpu.async_copy(total_hbm.at[pl.ds(base, _TPW)], ti, sem_o.at[0])
    cp4 = pltpu.async_copy(unit_hbm.at[pl.ds(base, _TPW)], ui, sem_o.at[0])
    cp1.wait()
    cp2.wait()
    cp3.wait()
    cp4.wait()

    def fire(w, s):
      t0 = w * _W
      pltpu.make_async_copy(
          dtab_hbm.at[di.at[pl.ds(t0, _W)]], ob.at[s], sem_g.at[s]).start()
      pltpu.make_async_copy(
          ttab_hbm.at[ti.at[pl.ds(t0, _W)]], bt.at[s], sem_g.at[s]).start()
      for g in range(_W // 16):
        src = t0 + g * 16
        d = dof[pl.ds(src, 16)]
        acc = jnp.zeros((16,), jnp.int32)
        for bound in _DOSE_BOUNDS:
          acc = acc + jnp.where(d > bound, 1, 0).astype(jnp.int32)
        ci[s, pl.ds(g * 16, 16)] = acc * 64 + ui[pl.ds(src, 16)]
      pltpu.make_async_copy(
          dutab_hbm.at[ci.at[s]], bdu.at[s], sem_g.at[s]).start()

    def wait_gathers(s):
      pltpu.make_async_copy(
          dtab_hbm.at[di.at[pl.ds(0, _W)]], ob.at[s], sem_g.at[s]).wait()
      pltpu.make_async_copy(
          ttab_hbm.at[ti.at[pl.ds(0, _W)]], bt.at[s], sem_g.at[s]).wait()
      pltpu.make_async_copy(
          dutab_hbm.at[ci.at[s]], bdu.at[s], sem_g.at[s]).wait()

    def drain_out(s):
      pltpu.make_async_copy(
          ob.at[s], out_hbm.at[pl.ds(base, _W)], sem_o.at[s]).wait()

    fire(0, 0)

    @pl.loop(0, _NWIN // 2)
    def _(p):
      for s in (0, 1):
        w = 2 * p + s
        nxt = 1 - s

        @pl.when(w + 1 < _NWIN)
        def _():
          @pl.when(w >= 1)
          def _():
            drain_out(nxt)
          fire(w + 1, nxt)

        wait_gathers(s)

        @pl.loop(0, _W)
        def _(r):
          for c in range(0, _D // 2, 16):
            slc = (s, r, pl.ds(c, 16))
            vo = plsc.bitcast(ob[slc], jnp.bfloat16)
            vdu = plsc.bitcast(bdu[slc], jnp.bfloat16)
            vt = plsc.bitcast(bt[slc], jnp.bfloat16)
            ob[slc] = plsc.bitcast(vo + vdu + vt, jnp.int32)

        pltpu.make_async_copy(
            ob.at[s], out_hbm.at[pl.ds(base + 2 * p * _W + s * _W, _W)],
            sem_o.at[s]).start()

    drain_out(0)
    drain_out(1)

  return k(data_i, dose_f, total_i, unit_i,
           data_table, du_table, total_table)


def kernel(data, dose, total, unit, data_table, dose_table, total_table,
           unit_table):
  du_table = _build_du_table(dose_table, unit_table)
  out = _composite_embedding(
      data.reshape(_N), dose.reshape(_N), total.reshape(_N), unit.reshape(_N),
      _pack_bf16(data_table.astype(jnp.bfloat16)), _pack_bf16(du_table),
      _pack_bf16(total_table.astype(jnp.bfloat16)))
  out = jax.lax.bitcast_convert_type(out, jnp.bfloat16).reshape(_N, _D)
  return out.astype(jnp.float32).reshape(_B, _T, _D)


# speedup vs baseline: 4.1821x; 2.7131x over previous
import dataclasses
import functools

import jax
import jax.numpy as jnp
from jax import lax
from jax.experimental import pallas as pl
from jax.experimental.pallas import tpu as pltpu
from jax.experimental.pallas import tpu_sc as plsc

_B, _T, _D = 4096, 50, 128
_N = _B * _T
_NWORKERS = 32
_TPW = _N // _NWORKERS
_W = 128
_NWIN = _TPW // _W

_DOSE_BOUNDS = (0.0, 0.1, 0.5, 1.0, 2.0, 5.0, 10.0, 20.0, 50.0, 100.0,
                200.0, 500.0, 1000.0)


def _build_du_table(dose_table, unit_table):
  nq, nu = dose_table.shape[0], unit_table.shape[0]

  def body(q_ref, u_ref, o_ref):
    o_ref[...] = (q_ref[...][:, None, :]
                  + u_ref[...][None, :, :]).astype(jnp.bfloat16)

  out = pl.pallas_call(
      body,
      out_shape=jax.ShapeDtypeStruct((nq, nu, _D), jnp.bfloat16),
  )(dose_table, unit_table)
  return out.reshape(nq * nu, _D)


def _pack_bf16(table):
  return jax.lax.bitcast_convert_type(
      jnp.stack([table[:, :_D // 2], table[:, _D // 2:]], axis=-1), jnp.int32)


def _composite_embedding(data_i, dose_f, total_i, unit_i,
                         data_table, du_table, total_table):
  mesh = plsc.VectorSubcoreMesh(core_axis_name="core",
                                subcore_axis_name="subcore")

  cparams = pltpu.CompilerParams()
  if "needs_layout_passes" in pltpu.CompilerParams.__dataclass_fields__:
    cparams = dataclasses.replace(cparams, needs_layout_passes=False)
  if "use_tc_tiling_on_sc" in pltpu.CompilerParams.__dataclass_fields__:
    cparams = dataclasses.replace(cparams, use_tc_tiling_on_sc=False)

  @functools.partial(
      pl.kernel,
      out_type=jax.ShapeDtypeStruct((_N, _D), jnp.float32),
      mesh=mesh,
      compiler_params=cparams,
      scratch_types=[
          pltpu.VMEM((_TPW,), jnp.int32),
          pltpu.VMEM((_TPW,), jnp.float32),
          pltpu.VMEM((_TPW,), jnp.int32),
          pltpu.VMEM((_TPW,), jnp.int32),
          pltpu.VMEM((2, _W), jnp.int32),
          pltpu.VMEM((2, _W, _D // 2), jnp.int32),
          pltpu.VMEM((2, _W, _D // 2), jnp.int32),
          pltpu.VMEM((2, _W, _D // 2), jnp.int32),
          pltpu.VMEM((2, _W, _D), jnp.float32),
          pltpu.SemaphoreType.DMA((2,)),
          pltpu.SemaphoreType.DMA((2,)),
      ],
  )
  def k(data_hbm, dose_hbm, total_hbm, unit_hbm,
        dtab_hbm, dutab_hbm, ttab_hbm,
        out_hbm, di, dof, ti, ui, ci, bd, bdu, bt, ob, sem_g, sem_o):
    wid = lax.axis_index("subcore") * 2 + lax.axis_index("core")
    base = wid * _TPW
    cp1 = pltpu.async_copy(data_hbm.at[pl.ds(base, _TPW)], di, sem_o.at[0])
    cp2 = pltpu.async_copy(dose_hbm.at[pl.ds(base, _TPW)], dof, sem_o.at[0])
    cp3 = pltpu.async_copy(total_hbm.at[pl.ds(base, _TPW)], ti, sem_o.at[0])
    cp4 = pltpu.async_copy(unit_hbm.at[pl.ds(base, _TPW)], ui, sem_o.at[0])
    cp1.wait()
    cp2.wait()
    cp3.wait()
    cp4.wait()

    def fire(w, s):
      t0 = w * _W
      pltpu.make_async_copy(
          dtab_hbm.at[di.at[pl.ds(t0, _W)]], bd.at[s], sem_g.at[s]).start()
      pltpu.make_async_copy(
          ttab_hbm.at[ti.at[pl.ds(t0, _W)]], bt.at[s], sem_g.at[s]).start()
      for g in range(_W // 16):
        src = t0 + g * 16
        d = dof[pl.ds(src, 16)]
        acc = jnp.zeros((16,), jnp.int32)
        for bound in _DOSE_BOUNDS:
          acc = acc + jnp.where(d > bound, 1, 0).astype(jnp.int32)
        ci[s, pl.ds(g * 16, 16)] = acc * 64 + ui[pl.ds(src, 16)]
      pltpu.make_async_copy(
          dutab_hbm.at[ci.at[s]], bdu.at[s], sem_g.at[s]).start()

    def wait_gathers(s):
      pltpu.make_async_copy(
          dtab_hbm.at[di.at[pl.ds(0, _W)]], bd.at[s], sem_g.at[s]).wait()
      pltpu.make_async_copy(
          ttab_hbm.at[ti.at[pl.ds(0, _W)]], bt.at[s], sem_g.at[s]).wait()
      pltpu.make_async_copy(
          dutab_hbm.at[ci.at[s]], bdu.at[s], sem_g.at[s]).wait()

    def drain_out(s):
      pltpu.make_async_copy(
          ob.at[s], out_hbm.at[pl.ds(base, _W)], sem_o.at[s]).wait()

    fire(0, 0)

    @pl.loop(0, _NWIN // 2)
    def _(p):
      for s in (0, 1):
        w = 2 * p + s
        nxt = 1 - s

        @pl.when(w + 1 < _NWIN)
        def _():
          @pl.when(w >= 1)
          def _():
            drain_out(nxt)
          fire(w + 1, nxt)

        wait_gathers(s)

        @pl.loop(0, _W)
        def _(r):
          for c in range(0, _D // 2, 16):
            slc = (s, r, pl.ds(c, 16))
            vd = plsc.bitcast(bd[slc], jnp.bfloat16)
            vdu = plsc.bitcast(bdu[slc], jnp.bfloat16)
            vt = plsc.bitcast(bt[slc], jnp.bfloat16)
            lo, hi = plsc.unpack(vd + vdu + vt,
                                 format=plsc.PackFormat.INTERLEAVED)
            ob[s, r, pl.ds(c, 16)] = lo
            ob[s, r, pl.ds(_D // 2 + c, 16)] = hi

        pltpu.make_async_copy(
            ob.at[s], out_hbm.at[pl.ds(base + 2 * p * _W + s * _W, _W)],
            sem_o.at[s]).start()

    drain_out(0)
    drain_out(1)

  return k(data_i, dose_f, total_i, unit_i,
           data_table, du_table, total_table)


def kernel(data, dose, total, unit, data_table, dose_table, total_table,
           unit_table):
  du_table = _build_du_table(dose_table, unit_table)
  out = _composite_embedding(
      data.reshape(_N), dose.reshape(_N), total.reshape(_N), unit.reshape(_N),
      _pack_bf16(data_table.astype(jnp.bfloat16)), _pack_bf16(du_table),
      _pack_bf16(total_table.astype(jnp.bfloat16)))
  return out.reshape(_B, _T, _D)


# speedup vs baseline: 5.6329x; 1.3469x over previous
import dataclasses
import functools

import jax
import jax.numpy as jnp
from jax import lax
from jax.experimental import pallas as pl
from jax.experimental.pallas import tpu as pltpu
from jax.experimental.pallas import tpu_sc as plsc

_B, _T, _D = 4096, 50, 128
_N = _B * _T
_NWORKERS = 32
_TPW = _N // _NWORKERS
_W = 128
_NWIN = _TPW // _W

_DOSE_BOUNDS = (0.0, 0.1, 0.5, 1.0, 2.0, 5.0, 10.0, 20.0, 50.0, 100.0,
                200.0, 500.0, 1000.0)


def _build_du_table(dose_table, unit_table):
  nq, nu = dose_table.shape[0], unit_table.shape[0]

  def body(q_ref, u_ref, o_ref):
    o_ref[...] = (q_ref[...][:, None, :]
                  + u_ref[...][None, :, :]).astype(jnp.bfloat16)

  out = pl.pallas_call(
      body,
      out_shape=jax.ShapeDtypeStruct((nq, nu, _D), jnp.bfloat16),
  )(dose_table, unit_table)
  return out.reshape(nq * nu, _D)


def _pack_bf16(table):
  return jax.lax.bitcast_convert_type(
      jnp.stack([table[:, :_D // 2], table[:, _D // 2:]], axis=-1), jnp.int32)


def _composite_embedding(data_i, dose_f, total_i, unit_i,
                         data_table, du_table, total_table):
  mesh = plsc.VectorSubcoreMesh(core_axis_name="core",
                                subcore_axis_name="subcore")

  cparams = pltpu.CompilerParams()
  if "needs_layout_passes" in pltpu.CompilerParams.__dataclass_fields__:
    cparams = dataclasses.replace(cparams, needs_layout_passes=False)
  if "use_tc_tiling_on_sc" in pltpu.CompilerParams.__dataclass_fields__:
    cparams = dataclasses.replace(cparams, use_tc_tiling_on_sc=False)

  @functools.partial(
      pl.kernel,
      out_type=jax.ShapeDtypeStruct((_N, _D), jnp.float32),
      mesh=mesh,
      compiler_params=cparams,
      scratch_types=[
          pltpu.VMEM((_TPW,), jnp.int32),
          pltpu.VMEM((_TPW,), jnp.float32),
          pltpu.VMEM((_TPW,), jnp.int32),
          pltpu.VMEM((_TPW,), jnp.int32),
          pltpu.VMEM((2, _W), jnp.int32),
          pltpu.VMEM((2, _W, _D // 2), jnp.int32),
          pltpu.VMEM((2, _W, _D // 2), jnp.int32),
          pltpu.VMEM((2, _W, _D), jnp.float32),
          pltpu.SemaphoreType.DMA((2,)),
          pltpu.SemaphoreType.DMA((2,)),
      ],
  )
  def k(data_hbm, dose_hbm, total_hbm, unit_hbm,
        dtab_hbm, dutab_hbm, ttab_hbm,
        out_hbm, di, dof, ti, ui, ci, bdu, bt, ob, sem_g, sem_o):
    wid = lax.axis_index("subcore") * 2 + lax.axis_index("core")
    base = wid * _TPW
    cp1 = pltpu.async_copy(data_hbm.at[pl.ds(base, _TPW)], di, sem_o.at[0])
    cp2 = pltpu.async_copy(dose_hbm.at[pl.ds(base, _TPW)], dof, sem_o.at[0])
    cp3 = pltpu.async_copy(total_hbm.at[pl.ds(base, _TPW)], ti, sem_o.at[0])
    cp4 = pltpu.async_copy(unit_hbm.at[pl.ds(base, _TPW)], ui, sem_o.at[0])
    cp1.wait()
    cp2.wait()
    cp3.wait()
    cp4.wait()

    def fire(w, s):
      t0 = w * _W
      pltpu.make_async_copy(
          dtab_hbm.at[di.at[pl.ds(t0, _W)]], ob.at[s], sem_g.at[s]).start()
      pltpu.make_async_copy(
          ttab_hbm.at[ti.at[pl.ds(t0, _W)]], bt.at[s], sem_g.at[s]).start()
      for g in range(_W // 16):
        src = t0 + g * 16
        d = dof[pl.ds(src, 16)]
        acc = jnp.zeros((16,), jnp.int32)
        for bound in _DOSE_BOUNDS:
          acc = acc + jnp.where(d > bound, 1, 0).astype(jnp.int32)
        ci[s, pl.ds(g * 16, 16)] = acc * 64 + ui[pl.ds(src, 16)]
      pltpu.make_async_copy(
          dutab_hbm.at[ci.at[s]], bdu.at[s], sem_g.at[s]).start()

    def wait_gathers(s):
      pltpu.make_async_copy(
          dtab_hbm.at[di.at[pl.ds(0, _W)]], ob.at[s], sem_g.at[s]).wait()
      pltpu.make_async_copy(
          ttab_hbm.at[ti.at[pl.ds(0, _W)]], bt.at[s], sem_g.at[s]).wait()
      pltpu.make_async_copy(
          dutab_hbm.at[ci.at[s]], bdu.at[s], sem_g.at[s]).wait()

    def drain_out(s):
      pltpu.make_async_copy(
          ob.at[s], out_hbm.at[pl.ds(base, _W)], sem_o.at[s]).wait()

    fire(0, 0)

    @pl.loop(0, _NWIN // 2)
    def _(p):
      for s in (0, 1):
        w = 2 * p + s
        nxt = 1 - s

        @pl.when(w + 1 < _NWIN)
        def _():
          @pl.when(w >= 1)
          def _():
            drain_out(nxt)
          fire(w + 1, nxt)

        wait_gathers(s)

        @pl.loop(0, _W)
        def _(r):
          for c in range(0, _D // 2, 16):
            slc = (s, r, pl.ds(c, 16))
            vdu = plsc.bitcast(bdu[slc], jnp.bfloat16)
            vt = plsc.bitcast(bt[slc], jnp.bfloat16)
            lo, hi = plsc.unpack(vdu + vt,
                                 format=plsc.PackFormat.INTERLEAVED)
            ob[slc] = ob[slc] + lo
            hslc = (s, r, pl.ds(_D // 2 + c, 16))
            ob[hslc] = ob[hslc] + hi

        pltpu.make_async_copy(
            ob.at[s], out_hbm.at[pl.ds(base + 2 * p * _W + s * _W, _W)],
            sem_o.at[s]).start()

    drain_out(0)
    drain_out(1)

  return k(data_i, dose_f, total_i, unit_i,
           data_table, du_table, total_table)


def kernel(data, dose, total, unit, data_table, dose_table, total_table,
           unit_table):
  du_table = _build_du_table(dose_table, unit_table)
  out = _composite_embedding(
      data.reshape(_N), dose.reshape(_N), total.reshape(_N), unit.reshape(_N),
      data_table, _pack_bf16(du_table),
      _pack_bf16(total_table.astype(jnp.bfloat16)))
  return out.reshape(_B, _T, _D)
